# baseline (speedup 1.0000x reference)
import functools

import jax
import jax.numpy as jnp
from jax import lax
from jax.experimental import pallas as pl
from jax.experimental.pallas import tpu as pltpu
from jax.experimental.pallas import tpu_sc as plsc

SIGMA = 0.2
ROWS, COLS, NIDX = 16384, 1024, 256
BLOCK_R = 1024
_LANES = 16


def _mask_sc_kernel(idx_hbm, mask_hbm, mask_v, zeros_v, idx_v):
    wid = lax.axis_index("s") * 2 + lax.axis_index("c")

    @pl.when(wid == 0)
    def _():
        for i in range(COLS // _LANES):
            mask_v[pl.ds(i * _LANES, _LANES)] = jnp.full(
                (_LANES,), 1.0, jnp.float32)
        for i in range(NIDX // _LANES):
            zeros_v[pl.ds(i * _LANES, _LANES)] = jnp.zeros(
                (_LANES,), jnp.float32)
        pltpu.sync_copy(idx_hbm, idx_v)
        pltpu.sync_copy(mask_v, mask_hbm)
        for i in range(NIDX // 128):
            pltpu.sync_copy(
                zeros_v.at[pl.ds(i * 128, 128)],
                mask_hbm.at[idx_v.at[pl.ds(i * 128, 128)]])


def _build_mask(idx):
    mesh = plsc.VectorSubcoreMesh(core_axis_name="c", subcore_axis_name="s")
    return pl.kernel(
        _mask_sc_kernel,
        mesh=mesh,
        out_type=jax.ShapeDtypeStruct((COLS,), jnp.float32),
        scratch_types=[
            pltpu.VMEM((COLS,), jnp.float32),
            pltpu.VMEM((NIDX,), jnp.float32),
            pltpu.VMEM((NIDX,), jnp.int32),
        ],
    )(idx)


def _dense_tc_kernel(mask_ref, x_ref, n_ref, o_ref):
    o_ref[...] = x_ref[...] * (1.0 + SIGMA * n_ref[...]) * mask_ref[...]


def kernel(x, noise, sample_index):
    idx = sample_index.astype(jnp.int32)
    mask = _build_mask(idx).reshape(1, COLS)
    return pl.pallas_call(
        _dense_tc_kernel,
        grid=(ROWS // BLOCK_R,),
        in_specs=[
            pl.BlockSpec((1, COLS), lambda i: (0, 0)),
            pl.BlockSpec((BLOCK_R, COLS), lambda i: (i, 0)),
            pl.BlockSpec((BLOCK_R, COLS), lambda i: (i, 0)),
        ],
        out_specs=pl.BlockSpec((BLOCK_R, COLS), lambda i: (i, 0)),
        out_shape=jax.ShapeDtypeStruct((ROWS, COLS), jnp.float32),
    )(mask, x, noise)


# speedup vs baseline: 1.3417x; 1.3417x over previous
import jax
import jax.numpy as jnp
from jax.experimental import pallas as pl
from jax.experimental.pallas import tpu as pltpu

SIGMA = 0.2
ROWS, COLS, NIDX = 16384, 1024, 256
BLOCK_R = 1024


def _fused_kernel(idx_ref, x_ref, n_ref, o_ref):
    cols = jax.lax.broadcasted_iota(jnp.int32, (NIDX, COLS), 1)
    hit = cols == idx_ref[...]
    mask = jnp.where(jnp.any(hit, axis=0, keepdims=True), 0.0, 1.0)
    o_ref[...] = x_ref[...] * (1.0 + SIGMA * n_ref[...]) * mask


def kernel(x, noise, sample_index):
    idx = sample_index.astype(jnp.int32).reshape(NIDX, 1)
    return pl.pallas_call(
        _fused_kernel,
        grid=(ROWS // BLOCK_R,),
        in_specs=[
            pl.BlockSpec((NIDX, 1), lambda i: (0, 0)),
            pl.BlockSpec((BLOCK_R, COLS), lambda i: (i, 0)),
            pl.BlockSpec((BLOCK_R, COLS), lambda i: (i, 0)),
        ],
        out_specs=pl.BlockSpec((BLOCK_R, COLS), lambda i: (i, 0)),
        out_shape=jax.ShapeDtypeStruct((ROWS, COLS), jnp.float32),
        compiler_params=pltpu.CompilerParams(
            dimension_semantics=("parallel",)),
    )(idx, x, noise)


# speedup vs baseline: 1.3776x; 1.0267x over previous
import jax
import jax.numpy as jnp
from jax.experimental import pallas as pl
from jax.experimental.pallas import tpu as pltpu

SIGMA = 0.2
ROWS, COLS, NIDX = 16384, 1024, 256
BLOCK_R = 1024


def _fused_kernel(idx_ref, x_ref, n_ref, o_ref, mask_ref):
    @pl.when(pl.program_id(0) == 0)
    def _build_mask():
        cols = jax.lax.broadcasted_iota(jnp.int32, (NIDX, COLS), 1)
        hit = cols == idx_ref[...]
        mask_ref[...] = jnp.where(jnp.any(hit, axis=0, keepdims=True), 0.0, 1.0)

    o_ref[...] = x_ref[...] * (1.0 + SIGMA * n_ref[...]) * mask_ref[...]


def kernel(x, noise, sample_index):
    idx = sample_index.astype(jnp.int32).reshape(NIDX, 1)
    return pl.pallas_call(
        _fused_kernel,
        grid=(ROWS // BLOCK_R,),
        in_specs=[
            pl.BlockSpec((NIDX, 1), lambda i: (0, 0)),
            pl.BlockSpec((BLOCK_R, COLS), lambda i: (i, 0)),
            pl.BlockSpec((BLOCK_R, COLS), lambda i: (i, 0)),
        ],
        out_specs=pl.BlockSpec((BLOCK_R, COLS), lambda i: (i, 0)),
        out_shape=jax.ShapeDtypeStruct((ROWS, COLS), jnp.float32),
        scratch_shapes=[pltpu.VMEM((1, COLS), jnp.float32)],
    )(idx, x, noise)
